# Initial kernel scaffold; baseline (speedup 1.0000x reference)
import jax
import jax.numpy as jnp
from jax.experimental import pallas as pl


def kernel(x, edge_index, edge_attr, grouping_matrices_true, W_embed, b_embed, W_gate, b_gate, W_self, W_msg, b_h, W_assign, b_assign):
    raise NotImplementedError("write your pallas kernel here")


# speedup vs baseline: 2.1683x; 2.1683x over previous
import functools

import jax
import jax.numpy as jnp
from jax import lax
from jax.experimental import pallas as pl
from jax.experimental.pallas import tpu as pltpu
from jax.experimental.pallas import tpu_sc as plsc

DF = 128
DE = 16
K = 64
LYR = 2

NC = 2
NS = 16
LANES = 16
NW = NC * NS
CHUNK = 128
IGRP = 16


def _embed_tc(x, W, b):
    n = x.shape[0]

    def body(x_ref, w_ref, b_ref, o_ref):
        o_ref[...] = (
            jnp.dot(x_ref[...], w_ref[...], preferred_element_type=jnp.float32)
            + b_ref[...]
        )

    return pl.pallas_call(
        body,
        out_shape=jax.ShapeDtypeStruct((n, DF), jnp.float32),
    )(x, W, b.reshape(1, DF))


def _gates_tc(ea_pad, W_gate, b_gate):
    e_pad = ea_pad.shape[0]
    blk = 4096
    nblk = e_pad // blk

    def body(ea_ref, wg_ref, bg_ref, o0_ref, o1_ref):
        ea = ea_ref[...]
        for l, o_ref in ((0, o0_ref), (1, o1_ref)):
            z = (
                jnp.dot(ea, wg_ref[l], preferred_element_type=jnp.float32)
                + bg_ref[l]
            )
            o_ref[...] = jax.nn.sigmoid(z)

    out = jax.ShapeDtypeStruct((e_pad, DF), jnp.float32)
    return pl.pallas_call(
        body,
        grid=(nblk,),
        in_specs=[
            pl.BlockSpec((blk, DE), lambda i: (i, 0)),
            pl.BlockSpec((LYR, DE, DF), lambda i: (0, 0, 0)),
            pl.BlockSpec((LYR, DF), lambda i: (0, 0)),
        ],
        out_specs=[
            pl.BlockSpec((blk, DF), lambda i: (i, 0)),
            pl.BlockSpec((blk, DF), lambda i: (i, 0)),
        ],
        out_shape=[out, out],
    )(ea_pad, W_gate, b_gate)


def _post_tc(h, agg, W_self_l, W_msg_l, b_h_l, W_assign_l, b_assign_l, g_true_l):
    n = h.shape[0]

    def body(h_ref, agg_ref, ws_ref, wm_ref, bh_ref, wa_ref, ba_ref, gt_ref,
             hn_ref, s_ref, loss_ref):
        a = agg_ref[0, :n, :] + agg_ref[1, :n, :]
        hn = jnp.dot(h_ref[...], ws_ref[...], preferred_element_type=jnp.float32)
        hn = hn + jnp.dot(a, wm_ref[...], preferred_element_type=jnp.float32)
        hn = jnp.maximum(hn + bh_ref[...], 0.0)
        hn_ref[...] = hn
        logits = (
            jnp.dot(hn, wa_ref[...], preferred_element_type=jnp.float32)
            + ba_ref[...]
        )
        m = jnp.max(logits, axis=-1, keepdims=True)
        e = jnp.exp(logits - m)
        s = e / jnp.sum(e, axis=-1, keepdims=True)
        s_ref[...] = s
        d = s - gt_ref[...]
        loss_ref[...] = jnp.reshape(jnp.sum(d * d) * (1.0 / (n * K)), (1, 1))

    return pl.pallas_call(
        body,
        out_shape=(
            jax.ShapeDtypeStruct((n, DF), jnp.float32),
            jax.ShapeDtypeStruct((n, K), jnp.float32),
            jax.ShapeDtypeStruct((1, 1), jnp.float32),
        ),
    )(h, agg, W_self_l, W_msg_l, b_h_l.reshape(1, DF), W_assign_l,
      b_assign_l.reshape(1, K), g_true_l)


def _sc_edge_layer(h, gate_l, src3, dst3, zeros_hbm, n_nodes, cpt):
    n_agg = zeros_hbm.shape[0]
    zrows = n_agg // NS
    mesh = plsc.VectorSubcoreMesh(core_axis_name="c", subcore_axis_name="s")

    @functools.partial(
        pl.kernel,
        out_type=jax.ShapeDtypeStruct((NC, n_agg, DF), jnp.float32),
        mesh=mesh,
        scratch_types=[
            pltpu.VMEM((IGRP, CHUNK), jnp.int32),
            pltpu.VMEM((IGRP, CHUNK), jnp.int32),
            pltpu.VMEM((CHUNK, DF), jnp.float32),
            pltpu.VMEM((CHUNK, DF), jnp.float32),
            pltpu.VMEM_SHARED((n_agg, DF), jnp.float32),
            pltpu.SemaphoreType.DMA,
            pltpu.SemaphoreType.DMA,
        ],
    )
    def sck(h_hbm, gate_hbm, src_hbm, dst_hbm, z_hbm, agg_hbm,
            src_v, dst_v, gate_v, rows_v, agg_sh, sem1, sem2):
        c = lax.axis_index("c")
        s = lax.axis_index("s")
        w = c * NS + s

        pltpu.sync_copy(z_hbm.at[pl.ds(s * zrows, zrows)],
                        agg_sh.at[pl.ds(s * zrows, zrows)])
        plsc.subcore_barrier()

        gate_base = w * (cpt * CHUNK)

        @pl.loop(0, cpt // IGRP)
        def _(g):
            cp_s = pltpu.async_copy(src_hbm.at[w, pl.ds(g * IGRP, IGRP)],
                                    src_v, sem1)
            cp_d = pltpu.async_copy(dst_hbm.at[w, pl.ds(g * IGRP, IGRP)],
                                    dst_v, sem2)
            cp_s.wait()
            cp_d.wait()

            @pl.loop(0, IGRP)
            def _(jj):
                j = g * IGRP + jj
                cp_g = pltpu.async_copy(
                    gate_hbm.at[pl.ds(gate_base + j * CHUNK, CHUNK)],
                    gate_v, sem1)
                cp_h = pltpu.async_copy(h_hbm.at[src_v.at[jj]], rows_v, sem2)
                cp_g.wait()
                cp_h.wait()

                @pl.loop(0, CHUNK)
                def _(r):
                    for q in range(DF // LANES):
                        sl = (r, pl.ds(q * LANES, LANES))
                        rows_v[sl] = rows_v[sl] * gate_v[sl]

                pltpu.sync_copy(rows_v, agg_sh.at[dst_v.at[jj]], add=True)

        plsc.subcore_barrier()
        pltpu.sync_copy(agg_sh.at[pl.ds(s * zrows, zrows)],
                        agg_hbm.at[c, pl.ds(s * zrows, zrows)])

    return sck(h, gate_l, src3, dst3, zeros_hbm)


def kernel(x, edge_index, edge_attr, grouping_matrices_true, W_embed, b_embed,
           W_gate, b_gate, W_self, W_msg, b_h, W_assign, b_assign):
    n = x.shape[0]
    e = edge_index.shape[1]

    tile_edges = CHUNK * NW
    cpt = -(-(-(-e // tile_edges)) // IGRP) * IGRP
    e_pad = tile_edges * cpt
    pad = e_pad - e

    src = jnp.concatenate([edge_index[0], jnp.zeros((pad,), jnp.int32)])
    dst = jnp.concatenate([edge_index[1], jnp.full((pad,), n, jnp.int32)])
    src3 = src.reshape(NW, cpt, CHUNK)
    dst3 = dst.reshape(NW, cpt, CHUNK)
    ea_pad = jnp.pad(edge_attr, ((0, pad), (0, 0)))

    n_agg = -(-n // (NS * 8)) * (NS * 8) + NS * 8
    zeros_hbm = jnp.zeros((n_agg, DF), jnp.float32)

    gate0, gate1 = _gates_tc(ea_pad, W_gate, b_gate)
    h0 = _embed_tc(x, W_embed, b_embed)

    agg0 = _sc_edge_layer(h0, gate0, src3, dst3, zeros_hbm, n, cpt)
    h1, s0, l0 = _post_tc(h0, agg0, W_self[0], W_msg[0], b_h[0],
                          W_assign[0], b_assign[0], grouping_matrices_true[0])
    agg1 = _sc_edge_layer(h1, gate1, src3, dst3, zeros_hbm, n, cpt)
    h2, s1, l1 = _post_tc(h1, agg1, W_self[1], W_msg[1], b_h[1],
                          W_assign[1], b_assign[1], grouping_matrices_true[1])

    return h2, jnp.stack([s0, s1]), jnp.stack([l0[0, 0], l1[0, 0]])


# speedup vs baseline: 2.1820x; 1.0063x over previous
import functools

import jax
import jax.numpy as jnp
from jax import lax
from jax.experimental import pallas as pl
from jax.experimental.pallas import tpu as pltpu
from jax.experimental.pallas import tpu_sc as plsc

DF = 128
DE = 16
K = 64
LYR = 2

NC = 2
NS = 16
LANES = 16
NW = NC * NS
CHUNK = 128
IGRP = 16


def _embed_tc(x, W, b):
    n = x.shape[0]

    def body(x_ref, w_ref, b_ref, o_ref):
        o_ref[...] = (
            jnp.dot(x_ref[...], w_ref[...], preferred_element_type=jnp.float32)
            + b_ref[...]
        )

    return pl.pallas_call(
        body,
        out_shape=jax.ShapeDtypeStruct((n, DF), jnp.float32),
    )(x, W, b.reshape(1, DF))


def _gates_tc(ea_pad, W_gate, b_gate):
    e_pad = ea_pad.shape[0]
    blk = 4096
    nblk = e_pad // blk

    def body(ea_ref, wg_ref, bg_ref, o0_ref, o1_ref):
        ea = ea_ref[...]
        for l, o_ref in ((0, o0_ref), (1, o1_ref)):
            z = (
                jnp.dot(ea, wg_ref[l], preferred_element_type=jnp.float32)
                + bg_ref[l]
            )
            o_ref[...] = jax.nn.sigmoid(z)

    out = jax.ShapeDtypeStruct((e_pad, DF), jnp.float32)
    return pl.pallas_call(
        body,
        grid=(nblk,),
        in_specs=[
            pl.BlockSpec((blk, DE), lambda i: (i, 0)),
            pl.BlockSpec((LYR, DE, DF), lambda i: (0, 0, 0)),
            pl.BlockSpec((LYR, DF), lambda i: (0, 0)),
        ],
        out_specs=[
            pl.BlockSpec((blk, DF), lambda i: (i, 0)),
            pl.BlockSpec((blk, DF), lambda i: (i, 0)),
        ],
        out_shape=[out, out],
    )(ea_pad, W_gate, b_gate)


def _post_tc(h, agg, W_self_l, W_msg_l, b_h_l, W_assign_l, b_assign_l, g_true_l):
    n = h.shape[0]

    def body(h_ref, agg_ref, ws_ref, wm_ref, bh_ref, wa_ref, ba_ref, gt_ref,
             hn_ref, s_ref, loss_ref):
        a = agg_ref[0, :n, :] + agg_ref[1, :n, :]
        hn = jnp.dot(h_ref[...], ws_ref[...], preferred_element_type=jnp.float32)
        hn = hn + jnp.dot(a, wm_ref[...], preferred_element_type=jnp.float32)
        hn = jnp.maximum(hn + bh_ref[...], 0.0)
        hn_ref[...] = hn
        logits = (
            jnp.dot(hn, wa_ref[...], preferred_element_type=jnp.float32)
            + ba_ref[...]
        )
        m = jnp.max(logits, axis=-1, keepdims=True)
        e = jnp.exp(logits - m)
        s = e / jnp.sum(e, axis=-1, keepdims=True)
        s_ref[...] = s
        d = s - gt_ref[...]
        loss_ref[...] = jnp.reshape(jnp.sum(d * d) * (1.0 / (n * K)), (1, 1))

    return pl.pallas_call(
        body,
        out_shape=(
            jax.ShapeDtypeStruct((n, DF), jnp.float32),
            jax.ShapeDtypeStruct((n, K), jnp.float32),
            jax.ShapeDtypeStruct((1, 1), jnp.float32),
        ),
    )(h, agg, W_self_l, W_msg_l, b_h_l.reshape(1, DF), W_assign_l,
      b_assign_l.reshape(1, K), g_true_l)


def _sc_edge_layer(h, gate_l, src3, dst3, zeros_hbm, n_nodes, cpt):
    n_agg = zeros_hbm.shape[0]
    zrows = n_agg // NS
    mesh = plsc.VectorSubcoreMesh(core_axis_name="c", subcore_axis_name="s")

    @functools.partial(
        pl.kernel,
        out_type=jax.ShapeDtypeStruct((NC, n_agg, DF), jnp.float32),
        mesh=mesh,
        scratch_types=[
            pltpu.VMEM((IGRP, CHUNK), jnp.int32),
            pltpu.VMEM((IGRP, CHUNK), jnp.int32),
            pltpu.VMEM((CHUNK, DF), jnp.float32),
            pltpu.VMEM((CHUNK, DF), jnp.float32),
            pltpu.VMEM_SHARED((n_agg, DF), jnp.float32),
            pltpu.SemaphoreType.DMA,
            pltpu.SemaphoreType.DMA,
        ],
    )
    def sck(h_hbm, gate_hbm, src_hbm, dst_hbm, z_hbm, agg_hbm,
            src_v, dst_v, gate_v, rows_v, agg_sh, sem1, sem2):
        c = lax.axis_index("c")
        s = lax.axis_index("s")
        w = c * NS + s

        pltpu.sync_copy(z_hbm.at[pl.ds(s * zrows, zrows)],
                        agg_sh.at[pl.ds(s * zrows, zrows)])
        plsc.subcore_barrier()

        gate_base = w * (cpt * CHUNK)

        @pl.loop(0, cpt // IGRP)
        def _(g):
            cp_s = pltpu.async_copy(src_hbm.at[w, pl.ds(g * IGRP, IGRP)],
                                    src_v, sem1)
            cp_d = pltpu.async_copy(dst_hbm.at[w, pl.ds(g * IGRP, IGRP)],
                                    dst_v, sem2)
            cp_s.wait()
            cp_d.wait()

            @pl.loop(0, IGRP)
            def _(jj):
                j = g * IGRP + jj
                cp_g = pltpu.async_copy(
                    gate_hbm.at[pl.ds(gate_base + j * CHUNK, CHUNK)],
                    gate_v, sem1)
                cp_h = pltpu.async_copy(h_hbm.at[src_v.at[jj]], rows_v, sem2)
                cp_g.wait()
                cp_h.wait()

                @pl.loop(0, CHUNK)
                def _(r):
                    for q in range(DF // LANES):
                        sl = (r, pl.ds(q * LANES, LANES))
                        rows_v[sl] = rows_v[sl] * gate_v[sl]

                pltpu.sync_copy(rows_v, agg_sh.at[dst_v.at[jj]], add=True)

        plsc.subcore_barrier()
        pltpu.sync_copy(agg_sh.at[pl.ds(s * zrows, zrows)],
                        agg_hbm.at[c, pl.ds(s * zrows, zrows)])

    return sck(h, gate_l, src3, dst3, zeros_hbm)


def kernel(x, edge_index, edge_attr, grouping_matrices_true, W_embed, b_embed,
           W_gate, b_gate, W_self, W_msg, b_h, W_assign, b_assign):
    n = x.shape[0]
    e = edge_index.shape[1]

    tile_edges = CHUNK * NW
    cpt = -(-(-(-e // tile_edges)) // IGRP) * IGRP
    e_pad = tile_edges * cpt
    pad = e_pad - e

    n_agg = -(-n // (NS * 8)) * (NS * 8) + NS * 8
    src = jnp.concatenate([edge_index[0], jnp.zeros((pad,), jnp.int32)])
    dst = jnp.concatenate(
        [edge_index[1], n + (jnp.arange(pad, dtype=jnp.int32) % (n_agg - n))])
    src3 = src.reshape(NW, cpt, CHUNK)
    dst3 = dst.reshape(NW, cpt, CHUNK)
    ea_pad = jnp.pad(edge_attr, ((0, pad), (0, 0)))

    zeros_hbm = jnp.zeros((n_agg, DF), jnp.float32)

    gate0, gate1 = _gates_tc(ea_pad, W_gate, b_gate)
    h0 = _embed_tc(x, W_embed, b_embed)

    agg0 = _sc_edge_layer(h0, gate0, src3, dst3, zeros_hbm, n, cpt)
    h1, s0, l0 = _post_tc(h0, agg0, W_self[0], W_msg[0], b_h[0],
                          W_assign[0], b_assign[0], grouping_matrices_true[0])
    agg1 = _sc_edge_layer(h1, gate1, src3, dst3, zeros_hbm, n, cpt)
    h2, s1, l1 = _post_tc(h1, agg1, W_self[1], W_msg[1], b_h[1],
                          W_assign[1], b_assign[1], grouping_matrices_true[1])

    return h2, jnp.stack([s0, s1]), jnp.stack([l0[0, 0], l1[0, 0]])
